# baseline (speedup 1.0000x reference)
import jax
import jax.numpy as jnp
from jax.experimental import pallas as pl
from jax.experimental.pallas import tpu as pltpu

N, D, E, H, TOPK = 2048, 768, 8, 1536, 2
BLK_N = 1024
NB = N // BLK_N


def _moe_body(x_ref, wg_ref, bg_ref, w1_ref, b1_ref, w2_ref, b2_ref,
              out_ref, gate_ref, acc_ref):
    e = pl.program_id(0)
    nb = pl.program_id(1)
    nsl = pl.ds(nb * BLK_N, BLK_N)

    @pl.when(e == 0)
    def _gate():
        xb = x_ref[...]
        logits = jnp.dot(xb, wg_ref[...],
                         preferred_element_type=jnp.float32) + bg_ref[...]
        m = jnp.max(logits, axis=-1, keepdims=True)
        p = jnp.exp(logits - m)
        p = p / jnp.sum(p, axis=-1, keepdims=True)
        eidx = jax.lax.broadcasted_iota(jnp.int32, p.shape, 1)
        big = jnp.int32(E)
        p1 = jnp.max(p, axis=-1, keepdims=True)
        i1 = jnp.min(jnp.where(p == p1, eidx, big), axis=-1, keepdims=True)
        mask1 = eidx == i1
        pm = jnp.where(mask1, -jnp.inf, p)
        p2 = jnp.max(pm, axis=-1, keepdims=True)
        i2 = jnp.min(jnp.where(pm == p2, eidx, big), axis=-1, keepdims=True)
        mask2 = eidx == i2
        denom = p1 + p2
        gate_ref[nsl, :] = jnp.where(mask1 | mask2, p / denom, 0.0)
        acc_ref[nsl, :] = xb

    xb16 = x_ref[...].astype(jnp.bfloat16)
    w1e = w1_ref[0].astype(jnp.bfloat16)
    w2e = w2_ref[0].astype(jnp.bfloat16)
    b1e = b1_ref[pl.ds(e, 1), :]
    b2e = b2_ref[pl.ds(e, 1), :]
    h = jnp.dot(xb16, w1e, preferred_element_type=jnp.float32) + b1e
    a = (0.5 * h * (1.0 + jax.lax.erf(h * 0.7071067811865476))
         ).astype(jnp.bfloat16)
    y = jnp.dot(a, w2e, preferred_element_type=jnp.float32) + b2e
    gate = gate_ref[nsl, :]
    col = jax.lax.broadcasted_iota(jnp.int32, gate.shape, 1)
    w_e = jnp.sum(jnp.where(col == e, gate, 0.0), axis=1, keepdims=True)
    acc_ref[nsl, :] += w_e * y

    @pl.when(e == E - 1)
    def _write():
        out_ref[...] = acc_ref[nsl, :]


@jax.jit
def kernel(x, Wg, bg, W1, b1, W2, b2):
    grid = (E, N // BLK_N)
    out = pl.pallas_call(
        _moe_body,
        grid=grid,
        in_specs=[
            pl.BlockSpec((BLK_N, D), lambda e, n: (n, 0)),
            pl.BlockSpec((D, E), lambda e, n: (0, 0)),
            pl.BlockSpec((E,), lambda e, n: (0,)),
            pl.BlockSpec((1, D, H), lambda e, n: (e, 0, 0)),
            pl.BlockSpec((E, H), lambda e, n: (0, 0)),
            pl.BlockSpec((1, H, D), lambda e, n: (e, 0, 0)),
            pl.BlockSpec((E, D), lambda e, n: (0, 0)),
        ],
        out_specs=pl.BlockSpec((BLK_N, D), lambda e, n: (n, 0)),
        out_shape=jax.ShapeDtypeStruct((N, D), jnp.float32),
        scratch_shapes=[
            pltpu.VMEM((N, E), jnp.float32),
            pltpu.VMEM((N, D), jnp.float32),
        ],
        compiler_params=pltpu.CompilerParams(
            dimension_semantics=("arbitrary", "arbitrary"),
        ),
    )(x, Wg, bg, W1, b1, W2, b2)
    return out
